# baseline (speedup 1.0000x reference)
import functools

import jax
import jax.numpy as jnp
from jax.experimental import pallas as pl
from jax.experimental.pallas import tpu as pltpu
from jax.experimental.pallas import tpu_sc as plsc

D = 64
PREP_BV = 512
GATHER_W = 128
N_CHUNKS = 8


def _prep_body(lo_ref, hi_ref, out_ref):
    out_ref[:, :D] = jnp.transpose(lo_ref[...], (1, 0))
    out_ref[:, D:] = jnp.transpose(hi_ref[...], (1, 0))


def _prep_table(tt, V):
    grid = pl.cdiv(V, 2 * PREP_BV)
    return pl.pallas_call(
        _prep_body,
        grid=(grid,),
        in_specs=[pl.BlockSpec((D, PREP_BV), lambda i: (0, 2 * i)),
                  pl.BlockSpec((D, PREP_BV), lambda i: (0, 2 * i + 1))],
        out_specs=pl.BlockSpec((PREP_BV, 2 * D), lambda i: (i, 0)),
        out_shape=jax.ShapeDtypeStruct((grid * PREP_BV, 2 * D), jnp.float32),
    )(tt, tt)


def _sc_gather(table_lin, idx_chunk, Tc):
    mesh = plsc.VectorSubcoreMesh(core_axis_name="core", subcore_axis_name="subcore")

    @functools.partial(
        pl.kernel,
        out_type=jax.ShapeDtypeStruct((Tc, D), jnp.float32),
        mesh=mesh,
        compiler_params=pltpu.CompilerParams(use_tc_tiling_on_sc=False),
    )
    def gather_kernel(tbl_hbm, idx_hbm, out_hbm):
        def body(i_vmem, o_vmem):
            pltpu.sync_copy(tbl_hbm.at[i_vmem.at[0]], o_vmem)

        pltpu.emit_pipeline(
            body,
            grid=(Tc // GATHER_W,),
            in_specs=[pl.BlockSpec((1, GATHER_W), index_map=lambda i: (0, i))],
            out_specs=[pl.BlockSpec((GATHER_W, D), index_map=lambda i: (i, 0))],
            core_axis_name=("core", "subcore"),
            dimension_semantics=(pltpu.PARALLEL,),
        )(idx_hbm, out_hbm)

    return gather_kernel(table_lin, idx_chunk).reshape(Tc // 2, 2 * D)


def _highway_body(emb_ref, w0, b0, w1, b1, _prev_ref, out_ref, *, Bh):
    e = emb_ref[...]
    h = jnp.transpose(e.astype(jnp.bfloat16), (1, 0))
    one = jnp.bfloat16(1.0)
    for (w, b) in ((w0, b0), (w1, b1)):
        pre = (jnp.dot(w[...], h, preferred_element_type=jnp.float32)
               + b[...]).astype(jnp.bfloat16)
        t = jnp.maximum(pre[:2 * D, :], jnp.bfloat16(0.0))
        g = jax.nn.sigmoid(pre[2 * D:, :])
        h = g * t + (one - g) * h
    out_ref[0, :, :Bh] = h[:D, :].astype(jnp.float32)
    out_ref[0, :, Bh:] = h[D:, :].astype(jnp.float32)


def _tc_highway_chunk(emb128, weights, prev_out, l0, L, B):
    rows = emb128.shape[0]
    Bh = B // 2
    steps = rows // Bh
    wspec = pl.BlockSpec((4 * D, 2 * D), lambda i: (0, 0))
    bspec = pl.BlockSpec((4 * D, 1), lambda i: (0, 0))
    kwargs = {}
    if prev_out is None:
        prev_out = jnp.zeros((1, 1), jnp.float32)
    else:
        kwargs["input_output_aliases"] = {5: 0}
    return pl.pallas_call(
        functools.partial(_highway_body, Bh=Bh),
        grid=(steps,),
        in_specs=[pl.BlockSpec((Bh, 2 * D), lambda i: (i, 0)),
                  wspec, bspec, wspec, bspec,
                  pl.BlockSpec(memory_space=pl.ANY)],
        out_specs=pl.BlockSpec((1, D, B), lambda i, l0=l0: (l0 + i, 0, 0)),
        out_shape=jax.ShapeDtypeStruct((L, D, B), jnp.float32),
        **kwargs,
    )(emb128, *weights, prev_out)


def kernel(batch, table, Wt0, bt0, Wg0, bg0, Wt1, bt1, Wg1, bg1):
    B, L = batch.shape
    T = B * L
    V = table.shape[0]
    Bh = B // 2
    Tc = T // N_CHUNKS
    Lc = L // N_CHUNKS

    bT = batch.T.astype(jnp.int32)
    iv = jnp.stack([bT[:, :Bh], bT[:, Bh:]], axis=2).reshape(L, B)
    q, rem = iv >> 10, iv & 1023
    idx = ((q << 10) | ((rem & 511) << 1) | (rem >> 9)).reshape(N_CHUNKS, 1, Tc)

    staged = _prep_table(table.T, V)
    table_lin = staged.reshape(2 * staged.shape[0], D)

    def stacked(Wt, bt, Wg, bg):
        wt, wg = Wt.T.astype(jnp.bfloat16), Wg.T.astype(jnp.bfloat16)
        z = jnp.zeros((D, D), jnp.bfloat16)
        w = jnp.block([[wt, z], [z, wt], [wg, z], [z, wg]])
        b = jnp.concatenate([bt, bt, bg, bg]).reshape(4 * D, 1)
        return w, b

    w0, b0 = stacked(Wt0, bt0, Wg0, bg0)
    w1, b1 = stacked(Wt1, bt1, Wg1, bg1)
    weights = (w0, b0, w1, b1)

    out = None
    for c in range(N_CHUNKS):
        emb128 = _sc_gather(table_lin, idx[c], Tc)
        out = _tc_highway_chunk(emb128, weights, out, c * Lc, L, B)
    return jnp.transpose(out, (2, 0, 1))


# speedup vs baseline: 2.1117x; 2.1117x over previous
import functools

import jax
import jax.numpy as jnp
from jax.experimental import pallas as pl
from jax.experimental.pallas import tpu as pltpu
from jax.experimental.pallas import tpu_sc as plsc

D = 64
PREP_BV = 4096
GATHER_W = 128
N_CHUNKS = 8


def _prep_body(tt_ref, out_ref):
    blk = tt_ref[...]
    out_ref[:, :D] = jnp.transpose(blk[:, :PREP_BV], (1, 0))
    out_ref[:, D:] = jnp.transpose(blk[:, PREP_BV:], (1, 0))


def _prep_table(tt, V):
    grid = pl.cdiv(V, 2 * PREP_BV)
    return pl.pallas_call(
        _prep_body,
        grid=(grid,),
        in_specs=[pl.BlockSpec((D, 2 * PREP_BV), lambda i: (0, i))],
        out_specs=pl.BlockSpec((PREP_BV, 2 * D), lambda i: (i, 0)),
        out_shape=jax.ShapeDtypeStruct((grid * PREP_BV, 2 * D), jnp.float32),
    )(tt)


def _sc_permute(idx_lin, perm, T):
    info = plsc.get_sparse_core_info()
    NW = info.num_cores * info.num_subcores
    per_w = T // NW
    mesh = plsc.VectorSubcoreMesh(core_axis_name="core", subcore_axis_name="subcore")

    @functools.partial(
        pl.kernel,
        out_type=jax.ShapeDtypeStruct((T,), jnp.int32),
        mesh=mesh,
        scratch_types=[pltpu.VMEM((per_w,), jnp.int32),
                       pltpu.VMEM((per_w,), jnp.int32),
                       pltpu.SemaphoreType.DMA],
        compiler_params=pltpu.CompilerParams(use_tc_tiling_on_sc=False),
    )
    def permute_kernel(src_hbm, perm_hbm, out_hbm, pvm, ivm, sem):
        wid = jax.lax.axis_index("subcore") * info.num_cores + jax.lax.axis_index("core")
        base = wid * per_w
        pltpu.sync_copy(perm_hbm.at[pl.ds(base, per_w)], pvm)
        pltpu.async_copy(src_hbm.at[pvm], ivm, sem).wait()
        pltpu.sync_copy(ivm, out_hbm.at[pl.ds(base, per_w)])

    return permute_kernel(idx_lin, perm)


def _sc_gather(table_lin, idx_chunk, Tc):
    mesh = plsc.VectorSubcoreMesh(core_axis_name="core", subcore_axis_name="subcore")

    @functools.partial(
        pl.kernel,
        out_type=jax.ShapeDtypeStruct((Tc, D), jnp.float32),
        mesh=mesh,
        compiler_params=pltpu.CompilerParams(use_tc_tiling_on_sc=False),
    )
    def gather_kernel(tbl_hbm, idx_hbm, out_hbm):
        def body(i_vmem, o_vmem):
            pltpu.sync_copy(tbl_hbm.at[i_vmem.at[0]], o_vmem)

        pltpu.emit_pipeline(
            body,
            grid=(Tc // GATHER_W,),
            in_specs=[pl.BlockSpec((1, GATHER_W), index_map=lambda i: (0, i))],
            out_specs=[pl.BlockSpec((GATHER_W, D), index_map=lambda i: (i, 0))],
            core_axis_name=("core", "subcore"),
            dimension_semantics=(pltpu.PARALLEL,),
        )(idx_hbm, out_hbm)

    return gather_kernel(table_lin, idx_chunk).reshape(Tc // 2, 2 * D)


def _highway_body(emb_ref, w0, b0, w1, b1, _prev_ref, out_ref, *, Bh):
    e = emb_ref[...]
    h = jnp.transpose(e.astype(jnp.bfloat16), (1, 0))
    one = jnp.bfloat16(1.0)
    for (w, b) in ((w0, b0), (w1, b1)):
        pre = (jnp.dot(w[...], h, preferred_element_type=jnp.float32)
               + b[...]).astype(jnp.bfloat16)
        t = jnp.maximum(pre[:2 * D, :], jnp.bfloat16(0.0))
        g = jax.nn.sigmoid(pre[2 * D:, :])
        h = g * t + (one - g) * h
    out_ref[0, :, :Bh] = h[:D, :].astype(jnp.float32)
    out_ref[0, :, Bh:] = h[D:, :].astype(jnp.float32)


def _tc_highway_chunk(emb128, weights, prev_out, l0, L, B):
    rows = emb128.shape[0]
    Bh = B // 2
    steps = rows // Bh
    wspec = pl.BlockSpec((4 * D, 2 * D), lambda i: (0, 0))
    bspec = pl.BlockSpec((4 * D, 1), lambda i: (0, 0))
    kwargs = {}
    if prev_out is None:
        prev_out = jnp.zeros((1, 1), jnp.float32)
    else:
        kwargs["input_output_aliases"] = {5: 0}
    return pl.pallas_call(
        functools.partial(_highway_body, Bh=Bh),
        grid=(steps,),
        in_specs=[pl.BlockSpec((Bh, 2 * D), lambda i: (i, 0)),
                  wspec, bspec, wspec, bspec,
                  pl.BlockSpec(memory_space=pl.ANY)],
        out_specs=pl.BlockSpec((1, D, B), lambda i, l0=l0: (l0 + i, 0, 0)),
        out_shape=jax.ShapeDtypeStruct((L, D, B), jnp.float32),
        **kwargs,
    )(emb128, *weights, prev_out)


def kernel(batch, table, Wt0, bt0, Wg0, bg0, Wt1, bt1, Wg1, bg1):
    B, L = batch.shape
    T = B * L
    V = table.shape[0]
    Bh = B // 2
    Tc = T // N_CHUNKS
    Lc = L // N_CHUNKS

    iv = batch.T.astype(jnp.int32).reshape(T)
    rem = iv & (2 * PREP_BV - 1)
    idx_lin = ((iv >> 13 << 13) | ((rem & (PREP_BV - 1)) << 1)
               | (rem >> 12))
    pos = jnp.arange(T, dtype=jnp.int32)
    perm = (pos & ~(B - 1)) | ((pos & 1) * Bh) | ((pos & (B - 1)) >> 1)
    idx = _sc_permute(idx_lin, perm, T).reshape(N_CHUNKS, 1, Tc)

    staged = _prep_table(table.T, V)
    table_lin = staged.reshape(2 * staged.shape[0], D)

    def stacked(Wt, bt, Wg, bg):
        wt, wg = Wt.T.astype(jnp.bfloat16), Wg.T.astype(jnp.bfloat16)
        z = jnp.zeros((D, D), jnp.bfloat16)
        w = jnp.block([[wt, z], [z, wt], [wg, z], [z, wg]])
        b = jnp.concatenate([bt, bt, bg, bg]).reshape(4 * D, 1)
        return w, b

    w0, b0 = stacked(Wt0, bt0, Wg0, bg0)
    w1, b1 = stacked(Wt1, bt1, Wg1, bg1)
    weights = (w0, b0, w1, b1)

    out = None
    for c in range(N_CHUNKS):
        emb128 = _sc_gather(table_lin, idx[c], Tc)
        out = _tc_highway_chunk(emb128, weights, out, c * Lc, L, B)
    return jnp.transpose(out, (2, 0, 1))


# speedup vs baseline: 2.1883x; 1.0363x over previous
import functools

import jax
import jax.numpy as jnp
from jax.experimental import pallas as pl
from jax.experimental.pallas import tpu as pltpu
from jax.experimental.pallas import tpu_sc as plsc

D = 64
PREP_BV = 8192
GATHER_W = 128
N_CHUNKS = 10


def _prep_body(tt_ref, out_ref):
    blk = tt_ref[...]
    out_ref[:, :D] = jnp.transpose(blk[:, :PREP_BV], (1, 0))
    out_ref[:, D:] = jnp.transpose(blk[:, PREP_BV:], (1, 0))


def _prep_table(tt, V):
    grid = pl.cdiv(V, 2 * PREP_BV)
    return pl.pallas_call(
        _prep_body,
        grid=(grid,),
        in_specs=[pl.BlockSpec((D, 2 * PREP_BV), lambda i: (0, i))],
        out_specs=pl.BlockSpec((PREP_BV, 2 * D), lambda i: (i, 0)),
        out_shape=jax.ShapeDtypeStruct((grid * PREP_BV, 2 * D), jnp.float32),
    )(tt)


def _sc_permute(idx_lin, perm, T):
    info = plsc.get_sparse_core_info()
    NW = info.num_cores * info.num_subcores
    per_w = T // NW
    mesh = plsc.VectorSubcoreMesh(core_axis_name="core", subcore_axis_name="subcore")

    @functools.partial(
        pl.kernel,
        out_type=jax.ShapeDtypeStruct((T,), jnp.int32),
        mesh=mesh,
        scratch_types=[pltpu.VMEM((per_w,), jnp.int32),
                       pltpu.VMEM((per_w,), jnp.int32),
                       pltpu.SemaphoreType.DMA],
        compiler_params=pltpu.CompilerParams(use_tc_tiling_on_sc=False),
    )
    def permute_kernel(src_hbm, perm_hbm, out_hbm, pvm, ivm, sem):
        wid = jax.lax.axis_index("subcore") * info.num_cores + jax.lax.axis_index("core")
        base = wid * per_w
        pltpu.sync_copy(perm_hbm.at[pl.ds(base, per_w)], pvm)
        pltpu.async_copy(src_hbm.at[pvm], ivm, sem).wait()
        pltpu.sync_copy(ivm, out_hbm.at[pl.ds(base, per_w)])

    return permute_kernel(idx_lin, perm)


def _sc_gather(table_lin, idx_chunk, Tc):
    mesh = plsc.VectorSubcoreMesh(core_axis_name="core", subcore_axis_name="subcore")

    @functools.partial(
        pl.kernel,
        out_type=jax.ShapeDtypeStruct((Tc, D), jnp.float32),
        mesh=mesh,
        compiler_params=pltpu.CompilerParams(use_tc_tiling_on_sc=False),
    )
    def gather_kernel(tbl_hbm, idx_hbm, out_hbm):
        def body(i_vmem, o_vmem):
            pltpu.sync_copy(tbl_hbm.at[i_vmem.at[0]], o_vmem)

        pltpu.emit_pipeline(
            body,
            grid=(Tc // GATHER_W,),
            in_specs=[pl.BlockSpec((1, GATHER_W), index_map=lambda i: (0, i))],
            out_specs=[pl.BlockSpec((GATHER_W, D), index_map=lambda i: (i, 0))],
            core_axis_name=("core", "subcore"),
            dimension_semantics=(pltpu.PARALLEL,),
        )(idx_hbm, out_hbm)

    return gather_kernel(table_lin, idx_chunk).reshape(Tc // 2, 2 * D)


def _highway_body(emb_ref, w0, b0, w1, b1, _prev_ref, out_ref, *, Bh):
    e = emb_ref[...]
    h = jnp.transpose(e.astype(jnp.bfloat16), (1, 0))
    one = jnp.bfloat16(1.0)
    for (w, b) in ((w0, b0), (w1, b1)):
        pre = (jnp.dot(w[...], h, preferred_element_type=jnp.float32)
               + b[...]).astype(jnp.bfloat16)
        t = jnp.maximum(pre[:2 * D, :], jnp.bfloat16(0.0))
        g = jax.nn.sigmoid(pre[2 * D:, :])
        h = g * t + (one - g) * h
    out_ref[0, :, :Bh] = h[:D, :].astype(jnp.float32)
    out_ref[0, :, Bh:] = h[D:, :].astype(jnp.float32)


def _tc_highway_chunk(emb128, weights, prev_out, l0, L, B):
    rows = emb128.shape[0]
    Bh = B // 2
    steps = rows // Bh
    wspec = pl.BlockSpec((4 * D, 2 * D), lambda i: (0, 0))
    bspec = pl.BlockSpec((4 * D, 1), lambda i: (0, 0))
    kwargs = {}
    if prev_out is None:
        prev_out = jnp.zeros((1, 1), jnp.float32)
    else:
        kwargs["input_output_aliases"] = {5: 0}
    return pl.pallas_call(
        functools.partial(_highway_body, Bh=Bh),
        grid=(steps,),
        in_specs=[pl.BlockSpec((Bh, 2 * D), lambda i: (i, 0)),
                  wspec, bspec, wspec, bspec,
                  pl.BlockSpec(memory_space=pl.ANY)],
        out_specs=pl.BlockSpec((1, D, B), lambda i, l0=l0: (l0 + i, 0, 0)),
        out_shape=jax.ShapeDtypeStruct((L, D, B), jnp.float32),
        **kwargs,
    )(emb128, *weights, prev_out)


def kernel(batch, table, Wt0, bt0, Wg0, bg0, Wt1, bt1, Wg1, bg1):
    B, L = batch.shape
    T = B * L
    V = table.shape[0]
    Bh = B // 2
    Tc = T // N_CHUNKS
    Lc = L // N_CHUNKS

    iv = batch.T.astype(jnp.int32).reshape(T)
    rem = iv & (2 * PREP_BV - 1)
    idx_lin = ((iv >> 14 << 14) | ((rem & (PREP_BV - 1)) << 1)
               | (rem >> 13))
    pos = jnp.arange(T, dtype=jnp.int32)
    perm = (pos & ~(B - 1)) | ((pos & 1) * Bh) | ((pos & (B - 1)) >> 1)
    idx = _sc_permute(idx_lin, perm, T).reshape(N_CHUNKS, 1, Tc)

    staged = _prep_table(table.T, V)
    table_lin = staged.reshape(2 * staged.shape[0], D)

    def stacked(Wt, bt, Wg, bg):
        wt, wg = Wt.T.astype(jnp.bfloat16), Wg.T.astype(jnp.bfloat16)
        z = jnp.zeros((D, D), jnp.bfloat16)
        w = jnp.block([[wt, z], [z, wt], [wg, z], [z, wg]])
        b = jnp.concatenate([bt, bt, bg, bg]).reshape(4 * D, 1)
        return w, b

    w0, b0 = stacked(Wt0, bt0, Wg0, bg0)
    w1, b1 = stacked(Wt1, bt1, Wg1, bg1)
    weights = (w0, b0, w1, b1)

    out = None
    for c in range(N_CHUNKS):
        emb128 = _sc_gather(table_lin, idx[c], Tc)
        out = _tc_highway_chunk(emb128, weights, out, c * Lc, L, B)
    return jnp.transpose(out, (2, 0, 1))
